# Initial kernel scaffold; baseline (speedup 1.0000x reference)
import jax
import jax.numpy as jnp
from jax.experimental import pallas as pl


def kernel(x, edge_index, W_l, b_l, W_r, training):
    raise NotImplementedError("write your pallas kernel here")


# speedup vs baseline: 5.6522x; 5.6522x over previous
import functools

import jax
import jax.numpy as jnp
from jax import lax
from jax.experimental import pallas as pl
from jax.experimental.pallas import tpu as pltpu
from jax.experimental.pallas import tpu_sc as plsc

N = 10000
E = 320000
D = 128
DA = 144
NC, NS = 2, 16
NW = NC * NS
EPT = E // NW
CH = 80
ROWS_PT = N // NS


def _sc_segment_sum(xa, src, dst, zrows):
    mesh = plsc.VectorSubcoreMesh(core_axis_name="c", subcore_axis_name="s")

    @functools.partial(
        pl.kernel,
        mesh=mesh,
        out_type=jax.ShapeDtypeStruct((NC, NS, ROWS_PT, DA), jnp.float32),
        scratch_types=[
            pltpu.VMEM((CH,), jnp.int32),
            pltpu.VMEM((CH,), jnp.int32),
            pltpu.VMEM((CH, DA), jnp.float32),
            pltpu.VMEM_SHARED((N, DA), jnp.float32),
            pltpu.SemaphoreType.DMA,
        ],
        compiler_params=pltpu.CompilerParams(use_tc_tiling_on_sc=False),
    )
    def k(xa_hbm, src_hbm, dst_hbm, z_hbm, part_hbm, src_v, dst_v, rows_v,
          acc_sh, sem):
        c = lax.axis_index("c")
        s = lax.axis_index("s")
        base = (c * NS + s) * EPT

        pltpu.sync_copy(z_hbm, acc_sh.at[pl.ds(s * ROWS_PT, ROWS_PT)])
        plsc.subcore_barrier()

        def body(i, carry):
            off = base + i * CH
            pltpu.sync_copy(src_hbm.at[pl.ds(off, CH)], src_v)
            pltpu.sync_copy(dst_hbm.at[pl.ds(off, CH)], dst_v)
            pltpu.async_copy(xa_hbm.at[src_v], rows_v, sem).wait()
            pltpu.sync_copy(rows_v, acc_sh.at[dst_v], add=True)
            return carry

        lax.fori_loop(0, EPT // CH, body, 0)

        plsc.subcore_barrier()
        pltpu.sync_copy(acc_sh.at[pl.ds(s * ROWS_PT, ROWS_PT)],
                        part_hbm.at[c, s])

    return k(xa, src, dst, zrows)


def _tc_finish(parts, x, wlt, wrt, b):
    B = 1000

    def body(p_ref, x_ref, wlt_ref, wrt_ref, b_ref, o_ref):
        p = p_ref[...]
        ssum = p[0] + p[1]
        summed = ssum[:, :D]
        cnt = jnp.sum(ssum[:, D:], axis=1, keepdims=True)
        mean = summed / jnp.maximum(cnt, 1.0)
        o_ref[...] = (
            jnp.dot(mean, wlt_ref[...], preferred_element_type=jnp.float32)
            + jnp.dot(x_ref[...], wrt_ref[...],
                      preferred_element_type=jnp.float32)
            + b_ref[...]
        )

    return pl.pallas_call(
        body,
        grid=(N // B,),
        in_specs=[
            pl.BlockSpec((NC, B, DA), lambda i: (0, i, 0)),
            pl.BlockSpec((B, D), lambda i: (i, 0)),
            pl.BlockSpec((D, D), lambda i: (0, 0)),
            pl.BlockSpec((D, D), lambda i: (0, 0)),
            pl.BlockSpec((1, D), lambda i: (0, 0)),
        ],
        out_specs=pl.BlockSpec((B, D), lambda i: (i, 0)),
        out_shape=jax.ShapeDtypeStruct((N, D), jnp.float32),
    )(parts, x, wlt, wrt, b)


def kernel(x, edge_index, W_l, b_l, W_r, training):
    xa = jnp.concatenate(
        [x, jnp.ones((N, 1), jnp.float32), jnp.zeros((N, DA - D - 1),
                                                     jnp.float32)], axis=1)
    src = edge_index[0].astype(jnp.int32)
    dst = edge_index[1].astype(jnp.int32)
    zrows = jnp.zeros((ROWS_PT, DA), jnp.float32)
    parts = _sc_segment_sum(xa, src, dst, zrows)
    parts = parts.reshape(NC, N, DA)
    return _tc_finish(parts, x, W_l.T, W_r.T, b_l.reshape(1, D))


# speedup vs baseline: 11.4690x; 2.0291x over previous
import functools

import jax
import jax.numpy as jnp
from jax import lax
from jax.experimental import pallas as pl
from jax.experimental.pallas import tpu as pltpu
from jax.experimental.pallas import tpu_sc as plsc

N = 10000
E = 320000
D = 128
DA = 144
NC, NS = 2, 16
NW = NC * NS
EPT = E // NW
CH = 80
NCHUNK = EPT // CH
NBUF = 3
G = 25
NSUP = NCHUNK // G
ROWS_PT = N // NS


def _sc_segment_sum(xa, srcr, dstr, zrows):
    mesh = plsc.VectorSubcoreMesh(core_axis_name="c", subcore_axis_name="s")

    @functools.partial(
        pl.kernel,
        mesh=mesh,
        out_type=jax.ShapeDtypeStruct((NC, NS, ROWS_PT, DA), jnp.float32),
        scratch_types=[
            pltpu.VMEM((G, CH), jnp.int32),
            pltpu.VMEM((G, CH), jnp.int32),
            pltpu.VMEM((NBUF, CH, DA), jnp.float32),
            pltpu.VMEM_SHARED((N, DA), jnp.float32),
        ] + [pltpu.SemaphoreType.DMA] * NBUF,
        compiler_params=pltpu.CompilerParams(use_tc_tiling_on_sc=False),
    )
    def k(xa_hbm, src_hbm, dst_hbm, z_hbm, part_hbm, src_v, dst_v, rows,
          acc_sh, *sems):
        c = lax.axis_index("c")
        s = lax.axis_index("s")
        wid = c * NS + s

        pltpu.sync_copy(z_hbm, acc_sh.at[pl.ds(s * ROWS_PT, ROWS_PT)])
        plsc.subcore_barrier()

        def superchunk(g5, carry):
            pltpu.sync_copy(src_hbm.at[wid, pl.ds(g5 * G, G)], src_v)
            pltpu.sync_copy(dst_hbm.at[wid, pl.ds(g5 * G, G)], dst_v)
            for b in range(NBUF):
                pltpu.async_copy(xa_hbm.at[src_v.at[b]], rows.at[b], sems[b])

            def chunk(i, carry2):
                for b in range(NBUF):

                    @pl.when(i % NBUF == b)
                    def _():
                        pltpu.make_async_copy(xa_hbm.at[src_v.at[i]],
                                              rows.at[b], sems[b]).wait()
                        pltpu.sync_copy(rows.at[b], acc_sh.at[dst_v.at[i]],
                                        add=True)

                        @pl.when(i + NBUF < G)
                        def _():
                            pltpu.async_copy(xa_hbm.at[src_v.at[i + NBUF]],
                                             rows.at[b], sems[b])
                return carry2

            lax.fori_loop(0, G, chunk, 0)
            return carry

        lax.fori_loop(0, NSUP, superchunk, 0)

        plsc.subcore_barrier()
        pltpu.sync_copy(acc_sh.at[pl.ds(s * ROWS_PT, ROWS_PT)],
                        part_hbm.at[c, s])

    return k(xa, srcr, dstr, zrows)


def _tc_finish(parts, x, wlt, wrt, b):
    B = 1000

    def body(p_ref, x_ref, wlt_ref, wrt_ref, b_ref, o_ref):
        p = p_ref[...]
        ssum = p[0] + p[1]
        summed = ssum[:, :D]
        cnt = jnp.sum(ssum[:, D:], axis=1, keepdims=True)
        mean = summed / jnp.maximum(cnt, 1.0)
        o_ref[...] = (
            jnp.dot(mean, wlt_ref[...], preferred_element_type=jnp.float32)
            + jnp.dot(x_ref[...], wrt_ref[...],
                      preferred_element_type=jnp.float32)
            + b_ref[...]
        )

    return pl.pallas_call(
        body,
        grid=(N // B,),
        in_specs=[
            pl.BlockSpec((NC, B, DA), lambda i: (0, i, 0)),
            pl.BlockSpec((B, D), lambda i: (i, 0)),
            pl.BlockSpec((D, D), lambda i: (0, 0)),
            pl.BlockSpec((D, D), lambda i: (0, 0)),
            pl.BlockSpec((1, D), lambda i: (0, 0)),
        ],
        out_specs=pl.BlockSpec((B, D), lambda i: (i, 0)),
        out_shape=jax.ShapeDtypeStruct((N, D), jnp.float32),
    )(parts, x, wlt, wrt, b)


def kernel(x, edge_index, W_l, b_l, W_r, training):
    xa = jnp.concatenate(
        [x, jnp.ones((N, 1), jnp.float32), jnp.zeros((N, DA - D - 1),
                                                     jnp.float32)], axis=1)
    src = edge_index[0].astype(jnp.int32).reshape(NW, NCHUNK, CH)
    dst = edge_index[1].astype(jnp.int32).reshape(NW, NCHUNK, CH)
    zrows = jnp.zeros((ROWS_PT, DA), jnp.float32)
    parts = _sc_segment_sum(xa, src, dst, zrows)
    parts = parts.reshape(NC, N, DA)
    return _tc_finish(parts, x, W_l.T, W_r.T, b_l.reshape(1, D))
